# baseline (speedup 1.0000x reference)
import jax
import jax.numpy as jnp
from jax import lax
from jax.experimental import pallas as pl
from jax.experimental.pallas import tpu as pltpu
from jax.experimental.pallas import tpu_sc as plsc

N = 1024
T = 3
E = 32768
M = N * N
NS = 16
EPW = E // NS
SLICE = M // NS
ROWS = EPW // 128
ZW = 8192
BM = 128


def _sc_scatter_body(idx_hbm, vals_hbm, zeros_hbm, ones_hbm, at_hbm, cnt_hbm,
                     acc, src_v, dst_v, idx_v, val_v, zer_v):
    c = lax.axis_index("c")
    s = lax.axis_index("s")
    base = s * EPW
    pltpu.sync_copy(zeros_hbm, zer_v)

    @pl.when(c == 1)
    def _():
        pltpu.sync_copy(ones_hbm, val_v)

    m_src = jnp.where(c == 0, 1, N).astype(jnp.int32)
    m_dst = jnp.where(c == 0, N, 1).astype(jnp.int32)

    for z in range(SLICE // ZW):
        pltpu.sync_copy(zer_v, acc.at[pl.ds(s * SLICE + z * ZW, ZW)])
    plsc.subcore_barrier()

    for t in range(T):
        pltpu.sync_copy(idx_hbm.at[pl.ds((t * 2 + 0) * E + base, EPW)], src_v)
        pltpu.sync_copy(idx_hbm.at[pl.ds((t * 2 + 1) * E + base, EPW)], dst_v)

        @pl.when(c == 0)
        def _():
            pltpu.sync_copy(vals_hbm.at[pl.ds(t * E + base, EPW)], val_v)

        for j in range(ROWS):
            def _idx_chunk(k, carry, j=j):
                off = j * 128 + k * 16
                sv = src_v[pl.ds(off, 16)]
                dv = dst_v[pl.ds(off, 16)]
                idx_v[j, pl.ds(k * 16, 16)] = sv * m_src + dv * m_dst
                return carry

            lax.fori_loop(0, 8, _idx_chunk, 0)
            pltpu.sync_copy(val_v.at[pl.ds(j * 128, 128)], acc.at[idx_v.at[j]],
                            add=True)
        plsc.subcore_barrier()

        @pl.when(c == 0)
        def _():
            pltpu.sync_copy(acc.at[pl.ds(s * SLICE, SLICE)],
                            at_hbm.at[pl.ds(t * M + s * SLICE, SLICE)])

        @pl.when(c == 1)
        def _():
            pltpu.sync_copy(acc.at[pl.ds(s * SLICE, SLICE)],
                            cnt_hbm.at[pl.ds(t * M + s * SLICE, SLICE)])

        if t < T - 1:
            plsc.subcore_barrier()
            for j in range(ROWS):
                pltpu.sync_copy(zer_v.at[pl.ds(0, 128)], acc.at[idx_v.at[j]])
            plsc.subcore_barrier()


def _sc_scatter(idx, vals, zeros, ones, interpret=False):
    mesh = plsc.VectorSubcoreMesh(core_axis_name="c", subcore_axis_name="s",
                                  num_cores=2, num_subcores=NS)
    f = pl.kernel(
        _sc_scatter_body,
        out_type=(jax.ShapeDtypeStruct((T * M,), jnp.float32),
                  jax.ShapeDtypeStruct((T * M,), jnp.float32)),
        mesh=mesh,
        scratch_types=[
            pltpu.VMEM_SHARED((M,), jnp.float32),
            pltpu.VMEM((EPW,), jnp.int32),
            pltpu.VMEM((EPW,), jnp.int32),
            pltpu.VMEM((ROWS, 128), jnp.int32),
            pltpu.VMEM((EPW,), jnp.float32),
            pltpu.VMEM((ZW,), jnp.float32),
        ],
        interpret=interpret,
    )
    return f(idx, vals, zeros, ones)


_PREC = lax.Precision.HIGHEST


def _encode_body(feats_ref, noise_ref, at_ref, wp_ref, bp_ref, wxh_ref,
                 wxn_ref, bx_ref, wm_ref, bm_ref, hn_ref):
    h = jnp.maximum(
        jnp.dot(feats_ref[0], wp_ref[...], preferred_element_type=jnp.float32)
        + bp_ref[...], 0.0)
    hx = jnp.dot(h, wxh_ref[...], preferred_element_type=jnp.float32)
    hx = hx + jnp.dot(noise_ref[0], wxn_ref[...],
                      preferred_element_type=jnp.float32)
    hx = jnp.maximum(hx + bx_ref[...], 0.0)
    agg = jnp.dot(at_ref[...].reshape(N, N), hx,
                  preferred_element_type=jnp.float32, precision=_PREC)
    hn = jnp.maximum(
        jnp.dot(agg, wm_ref[...], preferred_element_type=jnp.float32)
        + bm_ref[...], 0.0) + hx
    hn_ref[0] = hn


def _encode(feats3, noise, at3, W_proj, b_proj, W_xh, W_xn, b_x, W_m, b_m,
            interpret=False):
    return pl.pallas_call(
        _encode_body,
        grid=(T,),
        in_specs=[
            pl.BlockSpec((1, N, 128), lambda t: (t, 0, 0)),
            pl.BlockSpec((1, N, 16), lambda t: (t, 0, 0)),
            pl.BlockSpec((M,), lambda t: (t,)),
            pl.BlockSpec((128, 64), lambda t: (0, 0)),
            pl.BlockSpec((1, 64), lambda t: (0, 0)),
            pl.BlockSpec((64, 64), lambda t: (0, 0)),
            pl.BlockSpec((16, 64), lambda t: (0, 0)),
            pl.BlockSpec((1, 64), lambda t: (0, 0)),
            pl.BlockSpec((64, 64), lambda t: (0, 0)),
            pl.BlockSpec((1, 64), lambda t: (0, 0)),
        ],
        out_specs=pl.BlockSpec((1, N, 64), lambda t: (t, 0, 0)),
        out_shape=jax.ShapeDtypeStruct((T, N, 64), jnp.float32),
        interpret=interpret,
    )(feats3, noise, at3, W_proj, b_proj, W_xh, W_xn, b_x, W_m, b_m)


def _combine_body(params_ref, hn_ref, cnt0_ref, cnt1_ref, cnt2_ref, we_ref,
                  y_ref):
    i = pl.program_id(0)
    c1 = params_ref[0]
    c2 = params_ref[1]
    c3 = params_ref[2]
    k0 = params_ref[3]
    k1 = params_ref[4]
    k2 = params_ref[5]
    be = params_ref[6]
    w = we_ref[...]
    d = []
    for t, cnt_ref in enumerate((cnt0_ref, cnt1_ref, cnt2_ref)):
        hr = hn_ref[t, pl.ds(i * BM, BM), :]
        g = lax.dot_general(hr * w, hn_ref[t], (((1,), (1,)), ((), ())),
                            preferred_element_type=jnp.float32,
                            precision=_PREC)
        d.append(cnt_ref[...].reshape(BM, N) * (g + be))

    def silu(v):
        return v * jax.nn.sigmoid(v)

    xc0 = silu(c3 * d[0])
    xc1 = silu(c2 * d[0] + c3 * d[1])
    xc2 = silu(c1 * d[0] + c2 * d[1] + c3 * d[2])
    y_ref[...] = k0 * xc0 + k1 * xc1 + k2 * xc2


def _combine(params, hn, cnt_flat, w_edge2, interpret=False):
    nb = M // (BM * N)
    return pl.pallas_call(
        _combine_body,
        grid=(N // BM,),
        in_specs=[
            pl.BlockSpec(memory_space=pltpu.SMEM),
            pl.BlockSpec((T, N, 64), lambda i: (0, 0, 0)),
            pl.BlockSpec((BM * N,), lambda i: (0 * nb + i,)),
            pl.BlockSpec((BM * N,), lambda i: (1 * nb + i,)),
            pl.BlockSpec((BM * N,), lambda i: (2 * nb + i,)),
            pl.BlockSpec((1, 64), lambda i: (0, 0)),
        ],
        out_specs=pl.BlockSpec((BM, N), lambda i: (i, 0)),
        out_shape=jax.ShapeDtypeStruct((N, N), jnp.float32),
        interpret=interpret,
    )(params, hn, cnt_flat, cnt_flat, cnt_flat, w_edge2)


def kernel(feats, noise, sup_values, sup_indices, W_proj, b_proj, W_x, b_x,
           W_m, b_m, w_edge, b_edge, conv_w, A_log, dt, B_w, C_w, D_w):
    idx = sup_indices.astype(jnp.int32).reshape(T * 2 * E)
    zeros = jnp.zeros((ZW,), jnp.float32)
    ones = jnp.ones((EPW,), jnp.float32)
    at_flat, cnt_flat = _sc_scatter(idx, sup_values.reshape(T * E), zeros, ones)

    hn = _encode(feats[:T], noise, at_flat,
                 W_proj, b_proj.reshape(1, 64),
                 W_x[:64], W_x[64:], b_x.reshape(1, 64),
                 W_m, b_m.reshape(1, 64))

    dA = jnp.exp(-jnp.exp(A_log) * dt)
    bc = dt * jnp.dot(B_w, C_w)
    params = jnp.stack([conv_w[1], conv_w[2], conv_w[3],
                        bc * dA * dA, bc * dA, bc + D_w,
                        b_edge, jnp.float32(0.0)]).astype(jnp.float32)
    y = _combine(params, hn, cnt_flat, w_edge.reshape(1, 64))
    return (y,)


# speedup vs baseline: 1.0244x; 1.0244x over previous
import jax
import jax.numpy as jnp
from jax import lax
from jax.experimental import pallas as pl
from jax.experimental.pallas import tpu as pltpu
from jax.experimental.pallas import tpu_sc as plsc

N = 1024
T = 3
E = 32768
M = N * N
NS = 16
EPW = E // NS
SLICE = M // NS
ROWS = EPW // 128
ZW = 8192
BM = 128


def _sc_scatter_body(atidx_hbm, cntidx_hbm, vals_hbm, zeros_hbm, ones_hbm,
                     at_hbm, cnt_hbm, acc, idx_v, val_v, zer_v):
    c = lax.axis_index("c")
    s = lax.axis_index("s")
    base = s * EPW
    pltpu.sync_copy(zeros_hbm, zer_v)

    @pl.when(c == 1)
    def _():
        pltpu.sync_copy(ones_hbm, val_v)

    for z in range(SLICE // ZW):
        pltpu.sync_copy(zer_v, acc.at[pl.ds(s * SLICE + z * ZW, ZW)])
    plsc.subcore_barrier()

    for t in range(T):
        @pl.when(c == 0)
        def _():
            pltpu.sync_copy(atidx_hbm.at[pl.ds(t * E + base, EPW)], idx_v)
            pltpu.sync_copy(vals_hbm.at[pl.ds(t * E + base, EPW)], val_v)

        @pl.when(c == 1)
        def _():
            pltpu.sync_copy(cntidx_hbm.at[pl.ds(t * E + base, EPW)], idx_v)

        for j in range(ROWS):
            pltpu.sync_copy(val_v.at[pl.ds(j * 128, 128)],
                            acc.at[idx_v.at[pl.ds(j * 128, 128)]],
                            add=True)
        plsc.subcore_barrier()

        @pl.when(c == 0)
        def _():
            pltpu.sync_copy(acc.at[pl.ds(s * SLICE, SLICE)],
                            at_hbm.at[pl.ds(t * M + s * SLICE, SLICE)])

        @pl.when(c == 1)
        def _():
            pltpu.sync_copy(acc.at[pl.ds(s * SLICE, SLICE)],
                            cnt_hbm.at[pl.ds(t * M + s * SLICE, SLICE)])

        if t < T - 1:
            plsc.subcore_barrier()
            for j in range(ROWS):
                pltpu.sync_copy(zer_v.at[pl.ds(0, 128)],
                                acc.at[idx_v.at[pl.ds(j * 128, 128)]])
            plsc.subcore_barrier()


def _sc_scatter(at_idx, cnt_idx, vals, zeros, ones, interpret=False):
    mesh = plsc.VectorSubcoreMesh(core_axis_name="c", subcore_axis_name="s",
                                  num_cores=2, num_subcores=NS)
    f = pl.kernel(
        _sc_scatter_body,
        out_type=(jax.ShapeDtypeStruct((T * M,), jnp.float32),
                  jax.ShapeDtypeStruct((T * M,), jnp.float32)),
        mesh=mesh,
        scratch_types=[
            pltpu.VMEM_SHARED((M,), jnp.float32),
            pltpu.VMEM((EPW,), jnp.int32),
            pltpu.VMEM((EPW,), jnp.float32),
            pltpu.VMEM((ZW,), jnp.float32),
        ],
        interpret=interpret,
    )
    return f(at_idx, cnt_idx, vals, zeros, ones)


_PREC = lax.Precision.HIGHEST


def _encode_body(feats_ref, noise_ref, at_ref, wp_ref, bp_ref, wxh_ref,
                 wxn_ref, bx_ref, wm_ref, bm_ref, hn_ref):
    h = jnp.maximum(
        jnp.dot(feats_ref[0], wp_ref[...], preferred_element_type=jnp.float32)
        + bp_ref[...], 0.0)
    hx = jnp.dot(h, wxh_ref[...], preferred_element_type=jnp.float32)
    hx = hx + jnp.dot(noise_ref[0], wxn_ref[...],
                      preferred_element_type=jnp.float32)
    hx = jnp.maximum(hx + bx_ref[...], 0.0)
    agg = jnp.dot(at_ref[...].reshape(N, N), hx,
                  preferred_element_type=jnp.float32, precision=_PREC)
    hn = jnp.maximum(
        jnp.dot(agg, wm_ref[...], preferred_element_type=jnp.float32)
        + bm_ref[...], 0.0) + hx
    hn_ref[0] = hn


def _encode(feats3, noise, at3, W_proj, b_proj, W_xh, W_xn, b_x, W_m, b_m,
            interpret=False):
    return pl.pallas_call(
        _encode_body,
        grid=(T,),
        in_specs=[
            pl.BlockSpec((1, N, 128), lambda t: (t, 0, 0)),
            pl.BlockSpec((1, N, 16), lambda t: (t, 0, 0)),
            pl.BlockSpec((M,), lambda t: (t,)),
            pl.BlockSpec((128, 64), lambda t: (0, 0)),
            pl.BlockSpec((1, 64), lambda t: (0, 0)),
            pl.BlockSpec((64, 64), lambda t: (0, 0)),
            pl.BlockSpec((16, 64), lambda t: (0, 0)),
            pl.BlockSpec((1, 64), lambda t: (0, 0)),
            pl.BlockSpec((64, 64), lambda t: (0, 0)),
            pl.BlockSpec((1, 64), lambda t: (0, 0)),
        ],
        out_specs=pl.BlockSpec((1, N, 64), lambda t: (t, 0, 0)),
        out_shape=jax.ShapeDtypeStruct((T, N, 64), jnp.float32),
        interpret=interpret,
    )(feats3, noise, at3, W_proj, b_proj, W_xh, W_xn, b_x, W_m, b_m)


def _combine_body(params_ref, hn_ref, cnt0_ref, cnt1_ref, cnt2_ref, we_ref,
                  y_ref):
    i = pl.program_id(0)
    c1 = params_ref[0]
    c2 = params_ref[1]
    c3 = params_ref[2]
    k0 = params_ref[3]
    k1 = params_ref[4]
    k2 = params_ref[5]
    be = params_ref[6]
    w = we_ref[...]
    d = []
    for t, cnt_ref in enumerate((cnt0_ref, cnt1_ref, cnt2_ref)):
        hr = hn_ref[t, pl.ds(i * BM, BM), :]
        g = lax.dot_general(hr * w, hn_ref[t], (((1,), (1,)), ((), ())),
                            preferred_element_type=jnp.float32,
                            precision=_PREC)
        d.append(cnt_ref[...].reshape(BM, N) * (g + be))

    def silu(v):
        return v * jax.nn.sigmoid(v)

    xc0 = silu(c3 * d[0])
    xc1 = silu(c2 * d[0] + c3 * d[1])
    xc2 = silu(c1 * d[0] + c2 * d[1] + c3 * d[2])
    y_ref[...] = k0 * xc0 + k1 * xc1 + k2 * xc2


def _combine(params, hn, cnt_flat, w_edge2, interpret=False):
    nb = M // (BM * N)
    return pl.pallas_call(
        _combine_body,
        grid=(N // BM,),
        in_specs=[
            pl.BlockSpec(memory_space=pltpu.SMEM),
            pl.BlockSpec((T, N, 64), lambda i: (0, 0, 0)),
            pl.BlockSpec((BM * N,), lambda i: (0 * nb + i,)),
            pl.BlockSpec((BM * N,), lambda i: (1 * nb + i,)),
            pl.BlockSpec((BM * N,), lambda i: (2 * nb + i,)),
            pl.BlockSpec((1, 64), lambda i: (0, 0)),
        ],
        out_specs=pl.BlockSpec((BM, N), lambda i: (i, 0)),
        out_shape=jax.ShapeDtypeStruct((N, N), jnp.float32),
        interpret=interpret,
    )(params, hn, cnt_flat, cnt_flat, cnt_flat, w_edge2)


def kernel(feats, noise, sup_values, sup_indices, W_proj, b_proj, W_x, b_x,
           W_m, b_m, w_edge, b_edge, conv_w, A_log, dt, B_w, C_w, D_w):
    idx = sup_indices.astype(jnp.int32)
    src, dst = idx[:, 0, :], idx[:, 1, :]
    at_idx = (dst * N + src).reshape(T * E)
    cnt_idx = (src * N + dst).reshape(T * E)
    zeros = jnp.zeros((ZW,), jnp.float32)
    ones = jnp.ones((EPW,), jnp.float32)
    at_flat, cnt_flat = _sc_scatter(at_idx, cnt_idx,
                                    sup_values.reshape(T * E), zeros, ones)

    hn = _encode(feats[:T], noise, at_flat,
                 W_proj, b_proj.reshape(1, 64),
                 W_x[:64], W_x[64:], b_x.reshape(1, 64),
                 W_m, b_m.reshape(1, 64))

    dA = jnp.exp(-jnp.exp(A_log) * dt)
    bc = dt * jnp.dot(B_w, C_w)
    params = jnp.stack([conv_w[1], conv_w[2], conv_w[3],
                        bc * dA * dA, bc * dA, bc + D_w,
                        b_edge, jnp.float32(0.0)]).astype(jnp.float32)
    y = _combine(params, hn, cnt_flat, w_edge.reshape(1, 64))
    return (y,)


# speedup vs baseline: 1.0835x; 1.0578x over previous
import jax
import jax.numpy as jnp
from jax import lax
from jax.experimental import pallas as pl
from jax.experimental.pallas import tpu as pltpu
from jax.experimental.pallas import tpu_sc as plsc

N = 1024
T = 3
E = 32768
M = N * N
NS = 16
EPW = E // NS
SLICE = M // NS
ROWS = EPW // 128
ZW = 8192
BM = 128


def _sc_scatter_body(atidx_hbm, cntidx_hbm, vals_hbm, zeros_hbm, ones_hbm,
                     at_hbm, cnt_hbm, acc, idx_v, val_v, zer_v):
    c = lax.axis_index("c")
    s = lax.axis_index("s")
    base = s * EPW
    pltpu.sync_copy(zeros_hbm, zer_v)

    @pl.when(c == 1)
    def _():
        pltpu.sync_copy(ones_hbm, val_v)

    for z in range(SLICE // ZW):
        pltpu.sync_copy(zer_v, acc.at[pl.ds(s * SLICE + z * ZW, ZW)])
    plsc.subcore_barrier()

    for t in range(T):
        @pl.when(c == 0)
        def _():
            pltpu.sync_copy(atidx_hbm.at[pl.ds(t * E + base, EPW)], idx_v)
            pltpu.sync_copy(vals_hbm.at[pl.ds(t * E + base, EPW)], val_v)

        @pl.when(c == 1)
        def _():
            pltpu.sync_copy(cntidx_hbm.at[pl.ds(t * E + base, EPW)], idx_v)

        pltpu.sync_copy(val_v, acc.at[idx_v], add=True)
        plsc.subcore_barrier()

        @pl.when(c == 0)
        def _():
            pltpu.sync_copy(acc.at[pl.ds(s * SLICE, SLICE)],
                            at_hbm.at[pl.ds(t * M + s * SLICE, SLICE)])

        @pl.when(c == 1)
        def _():
            pltpu.sync_copy(acc.at[pl.ds(s * SLICE, SLICE)],
                            cnt_hbm.at[pl.ds(t * M + s * SLICE, SLICE)])

        if t < T - 1:
            plsc.subcore_barrier()
            pltpu.sync_copy(zer_v.at[pl.ds(0, EPW)], acc.at[idx_v])
            plsc.subcore_barrier()


def _sc_scatter(at_idx, cnt_idx, vals, zeros, ones, interpret=False):
    mesh = plsc.VectorSubcoreMesh(core_axis_name="c", subcore_axis_name="s",
                                  num_cores=2, num_subcores=NS)
    f = pl.kernel(
        _sc_scatter_body,
        out_type=(jax.ShapeDtypeStruct((T * M,), jnp.float32),
                  jax.ShapeDtypeStruct((T * M,), jnp.float32)),
        mesh=mesh,
        scratch_types=[
            pltpu.VMEM_SHARED((M,), jnp.float32),
            pltpu.VMEM((EPW,), jnp.int32),
            pltpu.VMEM((EPW,), jnp.float32),
            pltpu.VMEM((ZW,), jnp.float32),
        ],
        interpret=interpret,
    )
    return f(at_idx, cnt_idx, vals, zeros, ones)


_PREC = lax.Precision.HIGHEST


def _encode_body(feats_ref, noise_ref, at_ref, wp_ref, bp_ref, wxh_ref,
                 wxn_ref, bx_ref, wm_ref, bm_ref, hn_ref):
    h = jnp.maximum(
        jnp.dot(feats_ref[0], wp_ref[...], preferred_element_type=jnp.float32)
        + bp_ref[...], 0.0)
    hx = jnp.dot(h, wxh_ref[...], preferred_element_type=jnp.float32)
    hx = hx + jnp.dot(noise_ref[0], wxn_ref[...],
                      preferred_element_type=jnp.float32)
    hx = jnp.maximum(hx + bx_ref[...], 0.0)
    agg = jnp.dot(at_ref[...].reshape(N, N), hx,
                  preferred_element_type=jnp.float32, precision=_PREC)
    hn = jnp.maximum(
        jnp.dot(agg, wm_ref[...], preferred_element_type=jnp.float32)
        + bm_ref[...], 0.0) + hx
    hn_ref[0] = hn


def _encode(feats3, noise, at3, W_proj, b_proj, W_xh, W_xn, b_x, W_m, b_m,
            interpret=False):
    return pl.pallas_call(
        _encode_body,
        grid=(T,),
        in_specs=[
            pl.BlockSpec((1, N, 128), lambda t: (t, 0, 0)),
            pl.BlockSpec((1, N, 16), lambda t: (t, 0, 0)),
            pl.BlockSpec((M,), lambda t: (t,)),
            pl.BlockSpec((128, 64), lambda t: (0, 0)),
            pl.BlockSpec((1, 64), lambda t: (0, 0)),
            pl.BlockSpec((64, 64), lambda t: (0, 0)),
            pl.BlockSpec((16, 64), lambda t: (0, 0)),
            pl.BlockSpec((1, 64), lambda t: (0, 0)),
            pl.BlockSpec((64, 64), lambda t: (0, 0)),
            pl.BlockSpec((1, 64), lambda t: (0, 0)),
        ],
        out_specs=pl.BlockSpec((1, N, 64), lambda t: (t, 0, 0)),
        out_shape=jax.ShapeDtypeStruct((T, N, 64), jnp.float32),
        interpret=interpret,
    )(feats3, noise, at3, W_proj, b_proj, W_xh, W_xn, b_x, W_m, b_m)


def _combine_body(params_ref, hn_ref, cnt0_ref, cnt1_ref, cnt2_ref, we_ref,
                  y_ref):
    i = pl.program_id(0)
    c1 = params_ref[0]
    c2 = params_ref[1]
    c3 = params_ref[2]
    k0 = params_ref[3]
    k1 = params_ref[4]
    k2 = params_ref[5]
    be = params_ref[6]
    w = we_ref[...]
    d = []
    for t, cnt_ref in enumerate((cnt0_ref, cnt1_ref, cnt2_ref)):
        hr = hn_ref[t, pl.ds(i * BM, BM), :]
        g = lax.dot_general(hr * w, hn_ref[t], (((1,), (1,)), ((), ())),
                            preferred_element_type=jnp.float32,
                            precision=_PREC)
        d.append(cnt_ref[...].reshape(BM, N) * (g + be))

    def silu(v):
        return v * jax.nn.sigmoid(v)

    xc0 = silu(c3 * d[0])
    xc1 = silu(c2 * d[0] + c3 * d[1])
    xc2 = silu(c1 * d[0] + c2 * d[1] + c3 * d[2])
    y_ref[...] = k0 * xc0 + k1 * xc1 + k2 * xc2


def _combine(params, hn, cnt_flat, w_edge2, interpret=False):
    nb = M // (BM * N)
    return pl.pallas_call(
        _combine_body,
        grid=(N // BM,),
        in_specs=[
            pl.BlockSpec(memory_space=pltpu.SMEM),
            pl.BlockSpec((T, N, 64), lambda i: (0, 0, 0)),
            pl.BlockSpec((BM * N,), lambda i: (0 * nb + i,)),
            pl.BlockSpec((BM * N,), lambda i: (1 * nb + i,)),
            pl.BlockSpec((BM * N,), lambda i: (2 * nb + i,)),
            pl.BlockSpec((1, 64), lambda i: (0, 0)),
        ],
        out_specs=pl.BlockSpec((BM, N), lambda i: (i, 0)),
        out_shape=jax.ShapeDtypeStruct((N, N), jnp.float32),
        interpret=interpret,
    )(params, hn, cnt_flat, cnt_flat, cnt_flat, w_edge2)


def kernel(feats, noise, sup_values, sup_indices, W_proj, b_proj, W_x, b_x,
           W_m, b_m, w_edge, b_edge, conv_w, A_log, dt, B_w, C_w, D_w):
    idx = sup_indices.astype(jnp.int32)
    src, dst = idx[:, 0, :], idx[:, 1, :]
    at_idx = (dst * N + src).reshape(T * E)
    cnt_idx = (src * N + dst).reshape(T * E)
    zeros = jnp.zeros((ZW,), jnp.float32)
    ones = jnp.ones((EPW,), jnp.float32)
    at_flat, cnt_flat = _sc_scatter(at_idx, cnt_idx,
                                    sup_values.reshape(T * E), zeros, ones)

    hn = _encode(feats[:T], noise, at_flat,
                 W_proj, b_proj.reshape(1, 64),
                 W_x[:64], W_x[64:], b_x.reshape(1, 64),
                 W_m, b_m.reshape(1, 64))

    dA = jnp.exp(-jnp.exp(A_log) * dt)
    bc = dt * jnp.dot(B_w, C_w)
    params = jnp.stack([conv_w[1], conv_w[2], conv_w[3],
                        bc * dA * dA, bc * dA, bc + D_w,
                        b_edge, jnp.float32(0.0)]).astype(jnp.float32)
    y = _combine(params, hn, cnt_flat, w_edge.reshape(1, 64))
    return (y,)
